# baseline (speedup 1.0000x reference)
import functools

import jax
import jax.numpy as jnp
from jax import lax
from jax.experimental import pallas as pl
from jax.experimental.pallas import tpu as pltpu
from jax.experimental.pallas import tpu_sc as plsc

B = 512
N_ORB = 512
N_ELECS = 224
D = 288

NC = 2
NS = 16
NW = NC * NS
RPW = B // NW
L = 16
NCHUNK = N_ORB // L
IDX_HALF = N_ELECS // 2


def _make_sc_gather():
    mesh = plsc.VectorSubcoreMesh(core_axis_name="c", subcore_axis_name="s")

    @functools.partial(
        pl.kernel,
        mesh=mesh,
        out_type=jax.ShapeDtypeStruct((B, N_ELECS, D), jnp.float32),
        compiler_params=pltpu.CompilerParams(
            use_tc_tiling_on_sc=False, needs_layout_passes=False),
        scratch_types=[
            pltpu.VMEM((N_ORB,), jnp.float32),
            pltpu.VMEM((N_ELECS,), jnp.int32),
            pltpu.VMEM((N_ELECS, D), jnp.float32),
            pltpu.SemaphoreType.DMA,
        ],
    )
    def sc_gather(x_hbm, orb_hbm, out_hbm, x_v, idx_v, rows_v, sem):
        wid = lax.axis_index("s") * NC + lax.axis_index("c")
        base = wid * RPW
        col16 = lax.iota(jnp.int32, L)

        def row_body(r, carry):
            row = base + r
            pltpu.sync_copy(x_hbm.at[row], x_v)
            off = jnp.zeros((L,), jnp.int32)
            for j in range(NCHUNK):
                xc = x_v[pl.ds(j * L, L)]
                mask = xc != 0.0
                cum = plsc.cumsum(mask.astype(jnp.int32))
                pos = cum + off - 1
                vals = col16 + (j * L)
                plsc.store_scatter(idx_v, [pos], vals, mask=mask)
                off = off + plsc.all_reduce_population_count(mask)
            c0 = pltpu.async_copy(
                orb_hbm.at[idx_v.at[pl.ds(0, IDX_HALF)]],
                rows_v.at[pl.ds(0, IDX_HALF)], sem)
            c1 = pltpu.async_copy(
                orb_hbm.at[idx_v.at[pl.ds(IDX_HALF, IDX_HALF)]],
                rows_v.at[pl.ds(IDX_HALF, IDX_HALF)], sem)
            c0.wait()
            c1.wait()
            pltpu.sync_copy(rows_v, out_hbm.at[row])
            return carry

        lax.fori_loop(0, RPW, row_body, 0)

    return sc_gather


_sc_gather = _make_sc_gather()


def kernel(x, orbitals_mf, orbitals_hf):
    orbitals = jnp.concatenate((orbitals_mf, orbitals_hf), axis=1)
    return _sc_gather(x, orbitals)
